# baseline (speedup 1.0000x reference)
import functools

import jax
import jax.numpy as jnp
from jax import lax
from jax.experimental import pallas as pl
from jax.experimental.pallas import tpu as pltpu
from jax.experimental.pallas import tpu_sc as plsc

BATCH = 16384
EMB_K = 64
CHUNK = 128


def _gather_call(uidx2d, vidx2d, W, H):
  info = plsc.get_sparse_core_info()
  nc, ns = info.num_cores, info.num_subcores
  nw = nc * ns
  rows_per_w = BATCH // nw
  chunks_per_w = rows_per_w // CHUNK

  mesh = plsc.VectorSubcoreMesh(core_axis_name="c", subcore_axis_name="s")

  @functools.partial(
      pl.kernel,
      mesh=mesh,
      out_type=[
          jax.ShapeDtypeStruct((BATCH, EMB_K), jnp.float32),
          jax.ShapeDtypeStruct((BATCH, EMB_K), jnp.float32),
      ],
      scratch_types=[
          pltpu.VMEM((chunks_per_w, CHUNK), jnp.int32),
          pltpu.VMEM((chunks_per_w, CHUNK), jnp.int32),
          pltpu.VMEM((rows_per_w, EMB_K), jnp.float32),
          pltpu.VMEM((rows_per_w, EMB_K), jnp.float32),
          pltpu.SemaphoreType.DMA,
      ],
      compiler_params=pltpu.CompilerParams(use_tc_tiling_on_sc=False),
  )
  def gather_k(uidx_hbm, vidx_hbm, w_hbm, h_hbm, u_out, v_out,
               uidx_v, vidx_v, u_rows, v_rows, sem):
    wid = lax.axis_index("s") * nc + lax.axis_index("c")
    idx_base = wid * chunks_per_w
    pltpu.sync_copy(uidx_hbm.at[pl.ds(idx_base, chunks_per_w)], uidx_v)
    pltpu.sync_copy(vidx_hbm.at[pl.ds(idx_base, chunks_per_w)], vidx_v)
    copies = []
    for c in range(chunks_per_w):
      copies.append(pltpu.async_copy(
          w_hbm.at[uidx_v.at[c]], u_rows.at[pl.ds(c * CHUNK, CHUNK)], sem))
      copies.append(pltpu.async_copy(
          h_hbm.at[vidx_v.at[c]], v_rows.at[pl.ds(c * CHUNK, CHUNK)], sem))
    for cp in copies:
      cp.wait()
    row_base = wid * rows_per_w
    pltpu.sync_copy(u_rows, u_out.at[pl.ds(row_base, rows_per_w)])
    pltpu.sync_copy(v_rows, v_out.at[pl.ds(row_base, rows_per_w)])

  return gather_k(uidx2d, vidx2d, W, H)


def _mlp_body(u_ref, v_ref, w1a_ref, w1b_ref, b1_ref, w2_ref, b2_ref,
              w3_ref, b3_ref, out_ref):
  u = u_ref[...]
  v = v_ref[...]
  h = jnp.dot(u, w1a_ref[...], preferred_element_type=jnp.float32)
  h += jnp.dot(v, w1b_ref[...], preferred_element_type=jnp.float32)
  h = jnp.maximum(h + b1_ref[...], 0.0)
  h = jnp.dot(h, w2_ref[...], preferred_element_type=jnp.float32)
  h = jnp.maximum(h + b2_ref[...], 0.0)
  out_ref[...] = jnp.sum(h * w3_ref[...], axis=1) + b3_ref[0]


def _mlp_call(U, V, W1aT, W1bT, b1, W2T, b2, w3, b3):
  blk = 2048
  grid = (BATCH // blk,)
  full = lambda shape: pl.BlockSpec(shape, lambda i: (0,) * len(shape))
  return pl.pallas_call(
      _mlp_body,
      grid=grid,
      in_specs=[
          pl.BlockSpec((blk, EMB_K), lambda i: (i, 0)),
          pl.BlockSpec((blk, EMB_K), lambda i: (i, 0)),
          full((EMB_K, EMB_K)),
          full((EMB_K, EMB_K)),
          full((1, EMB_K)),
          full((EMB_K, EMB_K)),
          full((1, EMB_K)),
          full((1, EMB_K)),
          full((1,)),
      ],
      out_specs=pl.BlockSpec((blk,), lambda i: (i,)),
      out_shape=jax.ShapeDtypeStruct((BATCH,), jnp.float32),
  )(U, V, W1aT, W1bT, b1, W2T, b2, w3, b3)


@jax.jit
def kernel(x, W, H, W1, b1, W2, b2, W3, b3):
  uidx = x[:, 0].astype(jnp.int32).reshape(BATCH // CHUNK, CHUNK)
  vidx = x[:, 1].astype(jnp.int32).reshape(BATCH // CHUNK, CHUNK)
  U, V = _gather_call(uidx, vidx, W, H)
  out = _mlp_call(
      U, V,
      W1[:, :EMB_K].T, W1[:, EMB_K:].T, b1.reshape(1, EMB_K),
      W2.T, b2.reshape(1, EMB_K),
      W3.reshape(1, EMB_K), b3,
  )
  return out


# speedup vs baseline: 1.5203x; 1.5203x over previous
import functools

import jax
import jax.numpy as jnp
from jax import lax
from jax.experimental import pallas as pl
from jax.experimental.pallas import tpu as pltpu
from jax.experimental.pallas import tpu_sc as plsc

BATCH = 16384
EMB_K = 64
CHUNK = 128


def _gather_call(uidx, vidx, W, H):
  info = plsc.get_sparse_core_info()
  nc, ns, nl = info.num_cores, info.num_subcores, info.num_lanes
  nw = nc * ns
  rows_per_w = BATCH // nw
  half = rows_per_w // 2
  groups_per_half = half // nl

  mesh = plsc.VectorSubcoreMesh(core_axis_name="c", subcore_axis_name="s")

  @functools.partial(
      pl.kernel,
      mesh=mesh,
      out_type=[
          jax.ShapeDtypeStruct((BATCH, EMB_K), jnp.float32),
          jax.ShapeDtypeStruct((BATCH, EMB_K), jnp.float32),
      ],
      scratch_types=[
          pltpu.VMEM((rows_per_w,), jnp.int32),
          pltpu.VMEM((rows_per_w,), jnp.int32),
          pltpu.VMEM((half, EMB_K), jnp.float32),
          pltpu.VMEM((half, EMB_K), jnp.float32),
          pltpu.SemaphoreType.DMA,
      ],
  )
  def gather_k(uidx_hbm, vidx_hbm, w_hbm, h_hbm, u_out, v_out,
               uidx_v, vidx_v, u_rows, v_rows, sem):
    wid = lax.axis_index("s") * nc + lax.axis_index("c")
    row_base = wid * rows_per_w
    pltpu.sync_copy(uidx_hbm.at[pl.ds(row_base, rows_per_w)], uidx_v)
    pltpu.sync_copy(vidx_hbm.at[pl.ds(row_base, rows_per_w)], vidx_v)

    for h in range(2):
      def group(g, _, h=h):
        uvec = uidx_v[pl.ds(h * half + g * nl, nl)]
        vvec = vidx_v[pl.ds(h * half + g * nl, nl)]
        cps = []
        for l in range(nl):
          cps.append(pltpu.async_copy(
              w_hbm.at[pl.ds(uvec[l], 1)],
              u_rows.at[pl.ds(g * nl + l, 1)], sem))
          cps.append(pltpu.async_copy(
              h_hbm.at[pl.ds(vvec[l], 1)],
              v_rows.at[pl.ds(g * nl + l, 1)], sem))
        for cp in cps:
          cp.wait()
        return 0

      lax.fori_loop(0, groups_per_half, group, 0)
      pltpu.sync_copy(u_rows, u_out.at[pl.ds(row_base + h * half, half)])
      pltpu.sync_copy(v_rows, v_out.at[pl.ds(row_base + h * half, half)])

  return gather_k(uidx, vidx, W, H)


def _mlp_body(u_ref, v_ref, w1a_ref, w1b_ref, b1_ref, w2_ref, b2_ref,
              w3_ref, b3_ref, out_ref):
  u = u_ref[...]
  v = v_ref[...]
  h = jnp.dot(u, w1a_ref[...], preferred_element_type=jnp.float32)
  h += jnp.dot(v, w1b_ref[...], preferred_element_type=jnp.float32)
  h = jnp.maximum(h + b1_ref[...], 0.0)
  h = jnp.dot(h, w2_ref[...], preferred_element_type=jnp.float32)
  h = jnp.maximum(h + b2_ref[...], 0.0)
  out_ref[...] = jnp.sum(h * w3_ref[...], axis=1) + b3_ref[0]


def _mlp_call(U, V, W1aT, W1bT, b1, W2T, b2, w3, b3):
  blk = 2048
  grid = (BATCH // blk,)
  full = lambda shape: pl.BlockSpec(shape, lambda i: (0,) * len(shape))
  return pl.pallas_call(
      _mlp_body,
      grid=grid,
      in_specs=[
          pl.BlockSpec((blk, EMB_K), lambda i: (i, 0)),
          pl.BlockSpec((blk, EMB_K), lambda i: (i, 0)),
          full((EMB_K, EMB_K)),
          full((EMB_K, EMB_K)),
          full((1, EMB_K)),
          full((EMB_K, EMB_K)),
          full((1, EMB_K)),
          full((1, EMB_K)),
          full((1,)),
      ],
      out_specs=pl.BlockSpec((blk,), lambda i: (i,)),
      out_shape=jax.ShapeDtypeStruct((BATCH,), jnp.float32),
  )(U, V, W1aT, W1bT, b1, W2T, b2, w3, b3)


@jax.jit
def kernel(x, W, H, W1, b1, W2, b2, W3, b3):
  uidx = x[:, 0].astype(jnp.int32)
  vidx = x[:, 1].astype(jnp.int32)
  U, V = _gather_call(uidx, vidx, W, H)
  out = _mlp_call(
      U, V,
      W1[:, :EMB_K].T, W1[:, EMB_K:].T, b1.reshape(1, EMB_K),
      W2.T, b2.reshape(1, EMB_K),
      W3.reshape(1, EMB_K), b3,
  )
  return out
